# baseline (speedup 1.0000x reference)
import functools

import jax
import jax.numpy as jnp
from jax import lax
from jax.experimental import pallas as pl
from jax.experimental.pallas import tpu as pltpu
from jax.experimental.pallas import tpu_sc as plsc

B = 150000
VOCAB = 100000
H = 64

NC = 2
NS = 16
NW = NC * NS
L = 16

PER_W = 4704
B_PAD = NW * PER_W
CHUNK = 336
N_CHUNKS = PER_W // CHUNK


def _sc_body(wrd_hbm, cntxt_hbm, wemb_hbm, cemb_hbm, out_hbm,
             idx_w, idx_c, rows_w, rows_c, out_v, sem_w, sem_c):
  wid = lax.axis_index("s") * NC + lax.axis_index("c")
  wbase = wid * PER_W

  def chunk_body(ci, carry):
    base = pl.multiple_of(wbase + ci * CHUNK, 16)
    sl = pl.ds(base, CHUNK)
    pltpu.sync_copy(wrd_hbm.at[sl], idx_w)
    pltpu.sync_copy(cntxt_hbm.at[sl], idx_c)
    cp_w = pltpu.async_copy(wemb_hbm.at[idx_w], rows_w, sem_w)
    cp_c = pltpu.async_copy(cemb_hbm.at[idx_c], rows_c, sem_c)
    cp_w.wait()
    cp_c.wait()

    def group_body(g, carry2):
      rid = g * L + lax.iota(jnp.int32, L)
      acc = jnp.zeros((L,), jnp.float32)
      for j in range(H):
        cj = jnp.full((L,), j, jnp.int32)
        vw = plsc.load_gather(rows_w, [rid, cj])
        vc = plsc.load_gather(rows_c, [rid, cj])
        acc = acc + vw * vc
      out_v[pl.ds(g * L, L)] = 1.0 / (1.0 + jnp.exp(-acc))
      return carry2

    lax.fori_loop(0, CHUNK // L, group_body, 0)
    pltpu.sync_copy(out_v, out_hbm.at[sl])
    return carry

  lax.fori_loop(0, N_CHUNKS, chunk_body, 0)


@jax.jit
def _run(wrd_flat, cntxt_flat, word_emb, context_emb):
  mesh = plsc.VectorSubcoreMesh(core_axis_name="c", subcore_axis_name="s")
  k = functools.partial(
      pl.kernel,
      out_type=jax.ShapeDtypeStruct((B_PAD,), jnp.float32),
      mesh=mesh,
      compiler_params=pltpu.CompilerParams(
          needs_layout_passes=False, use_tc_tiling_on_sc=False),
      scratch_types=[
          pltpu.VMEM((CHUNK,), jnp.int32),
          pltpu.VMEM((CHUNK,), jnp.int32),
          pltpu.VMEM((CHUNK, H), jnp.float32),
          pltpu.VMEM((CHUNK, H), jnp.float32),
          pltpu.VMEM((CHUNK,), jnp.float32),
          pltpu.SemaphoreType.DMA,
          pltpu.SemaphoreType.DMA,
      ],
  )(_sc_body)
  return k(wrd_flat, cntxt_flat, word_emb, context_emb)


def kernel(wrd, cntxt, word_emb, context_emb):
  wrd_flat = jnp.pad(wrd.reshape(-1).astype(jnp.int32), (0, B_PAD - B))
  cntxt_flat = jnp.pad(cntxt.reshape(-1).astype(jnp.int32), (0, B_PAD - B))
  out = _run(wrd_flat, cntxt_flat,
             word_emb.astype(jnp.float32), context_emb.astype(jnp.float32))
  return out[:B].reshape(B, 1)


# speedup vs baseline: 2.1918x; 2.1918x over previous
import functools

import jax
import jax.numpy as jnp
from jax import lax
from jax.experimental import pallas as pl
from jax.experimental.pallas import tpu as pltpu
from jax.experimental.pallas import tpu_sc as plsc

B = 150000
VOCAB = 100000
H = 64

NC = 2
NS = 16
NW = NC * NS
L = 16

PER_W = 4688
CHUNK = 336
N_CHUNKS = -(-PER_W // CHUNK)


def _sc_body(wrd_hbm, cntxt_hbm, wemb_hbm, cemb_hbm, out_hbm,
             idx_w, idx_c, rows_w, rows_c, out_v, sem_w, sem_c):
  wid = lax.axis_index("s") * NC + lax.axis_index("c")
  wstart = wid * PER_W
  wlast = jnp.minimum(wstart + PER_W, B) - CHUNK
  lane = lax.iota(jnp.int32, L)

  def chunk_body(ci, carry):
    base = pl.multiple_of(jnp.minimum(wstart + ci * CHUNK, wlast), 16)
    sl = pl.ds(base, CHUNK)
    pltpu.sync_copy(wrd_hbm.at[sl], idx_w)
    pltpu.sync_copy(cntxt_hbm.at[sl], idx_c)
    cp_w = pltpu.async_copy(wemb_hbm.at[idx_w], rows_w, sem_w)
    cp_c = pltpu.async_copy(cemb_hbm.at[idx_c], rows_c, sem_c)
    cp_w.wait()
    cp_c.wait()

    def group_body(g, carry2):
      rid = g * L + lane
      acc = jnp.zeros((L,), jnp.float32)
      for j in range(H):
        cj = lax.bitwise_and(lane + j, H - 1)
        vw = plsc.load_gather(rows_w, [rid, cj])
        vc = plsc.load_gather(rows_c, [rid, cj])
        acc = acc + vw * vc
      out_v[pl.ds(g * L, L)] = 1.0 / (1.0 + jnp.exp(-acc))
      return carry2

    lax.fori_loop(0, CHUNK // L, group_body, 0)
    pltpu.sync_copy(out_v, out_hbm.at[sl])
    return carry

  lax.fori_loop(0, N_CHUNKS, chunk_body, 0)


@jax.jit
def _run(wrd_flat, cntxt_flat, word_emb, context_emb):
  mesh = plsc.VectorSubcoreMesh(core_axis_name="c", subcore_axis_name="s")
  k = functools.partial(
      pl.kernel,
      out_type=jax.ShapeDtypeStruct((B,), jnp.float32),
      mesh=mesh,
      compiler_params=pltpu.CompilerParams(
          needs_layout_passes=False, use_tc_tiling_on_sc=False),
      scratch_types=[
          pltpu.VMEM((CHUNK,), jnp.int32),
          pltpu.VMEM((CHUNK,), jnp.int32),
          pltpu.VMEM((CHUNK, H), jnp.float32),
          pltpu.VMEM((CHUNK, H), jnp.float32),
          pltpu.VMEM((CHUNK,), jnp.float32),
          pltpu.SemaphoreType.DMA,
          pltpu.SemaphoreType.DMA,
      ],
  )(_sc_body)
  return k(wrd_flat, cntxt_flat, word_emb, context_emb)


def kernel(wrd, cntxt, word_emb, context_emb):
  wrd_flat = wrd.reshape(-1).astype(jnp.int32)
  cntxt_flat = cntxt.reshape(-1).astype(jnp.int32)
  out = _run(wrd_flat, cntxt_flat,
             word_emb.astype(jnp.float32), context_emb.astype(jnp.float32))
  return out.reshape(B, 1)
